# Initial kernel scaffold; baseline (speedup 1.0000x reference)
import jax
import jax.numpy as jnp
from jax.experimental import pallas as pl


def kernel(x, edge_index, Wl0, Wr0, b0, Wl1, Wr1, b1, Wl2, Wr2, b2):
    raise NotImplementedError("write your pallas kernel here")


# speedup vs baseline: 6.0727x; 6.0727x over previous
import functools

import jax
import jax.numpy as jnp
from jax import lax
from jax.experimental import pallas as pl
from jax.experimental.pallas import tpu as pltpu
from jax.experimental.pallas import tpu_sc as plsc

N = 100000
E = 1600000
D = 32
H = 16

NC = 2
NS = 16

CHUNK = 128
GROUP = 32
TILE_E = 102400
E_PAD = NS * TILE_E
GROUPS = TILE_E // (GROUP * CHUNK)

ACC_ROWS = 100864
ZROWS = 788

def _sc_segsum(ua, ub, src2, dst2, zeros2, zeros1, with_deg):
    out_type = [
        jax.ShapeDtypeStruct((ACC_ROWS, H), jnp.float32),
        jax.ShapeDtypeStruct((ACC_ROWS, H), jnp.float32),
    ]
    scratch = [
        pltpu.VMEM_SHARED((ACC_ROWS, H), jnp.float32),
        pltpu.VMEM((GROUP, CHUNK), jnp.int32),
        pltpu.VMEM((GROUP, CHUNK), jnp.int32),
        pltpu.VMEM((CHUNK, H), jnp.float32),
        pltpu.SemaphoreType.DMA,
    ]
    if with_deg:
        out_type.append(jax.ShapeDtypeStruct((NS, ACC_ROWS // NS), jnp.float32))
        scratch += [
            pltpu.VMEM_SHARED((ACC_ROWS,), jnp.float32),
            pltpu.VMEM((CHUNK,), jnp.float32),
        ]

    mesh = plsc.VectorSubcoreMesh(core_axis_name="c", subcore_axis_name="s")

    def body(ua_hbm, ub_hbm, src_hbm, dst_hbm, z2_hbm, z1_hbm,
             oa_hbm, ob_hbm, *rest):
        if with_deg:
            deg_hbm, acc, idx_s, idx_d, rows, gsem, deg_acc, ones = rest
        else:
            acc, idx_s, idx_d, rows, gsem = rest
        c = lax.axis_index("c")
        s = lax.axis_index("s")

        nrows = ACC_ROWS // NS
        zbase = s * nrows
        pltpu.sync_copy(z2_hbm.at[pl.ds(zbase, nrows)],
                        acc.at[pl.ds(zbase, nrows)])
        if with_deg:
            pltpu.sync_copy(z1_hbm.at[pl.ds(zbase, nrows)],
                            deg_acc.at[pl.ds(zbase, nrows)])
            def fo(i, _):
                ones[pl.ds(i * 16, 16)] = jnp.ones((16,), jnp.float32)
                return 0
            lax.fori_loop(0, CHUNK // 16, fo, 0)
        plsc.subcore_barrier()

        rows_per_tile = TILE_E // CHUNK
        def group_body(g, _):
            r0 = s * rows_per_tile + g * GROUP
            pltpu.sync_copy(src_hbm.at[pl.ds(r0, GROUP)], idx_s)
            pltpu.sync_copy(dst_hbm.at[pl.ds(r0, GROUP)], idx_d)
            for j in range(GROUP):
                @pl.when(c == 0)
                def _():
                    pltpu.async_copy(ua_hbm.at[idx_s.at[j]], rows, gsem).wait()
                @pl.when(c == 1)
                def _():
                    pltpu.async_copy(ub_hbm.at[idx_s.at[j]], rows, gsem).wait()
                pltpu.sync_copy(rows, acc.at[idx_d.at[j]], add=True)
                if with_deg:
                    @pl.when(c == 0)
                    def _():
                        pltpu.sync_copy(ones, deg_acc.at[idx_d.at[j]], add=True)
            return 0
        lax.fori_loop(0, GROUPS, group_body, 0)

        plsc.subcore_barrier()
        @pl.when(c == 0)
        def _():
            pltpu.sync_copy(acc.at[pl.ds(zbase, nrows)],
                            oa_hbm.at[pl.ds(zbase, nrows)])
            if with_deg:
                pltpu.sync_copy(deg_acc.at[pl.ds(zbase, nrows)],
                                deg_hbm.at[s])
        @pl.when(c == 1)
        def _():
            pltpu.sync_copy(acc.at[pl.ds(zbase, nrows)],
                            ob_hbm.at[pl.ds(zbase, nrows)])

    f = pl.kernel(body, out_type=tuple(out_type), mesh=mesh,
                  scratch_types=tuple(scratch),
                  compiler_params=pltpu.CompilerParams(use_tc_tiling_on_sc=False))
    return f(ua, ub, src2, dst2, zeros2, zeros1)


BN = 2000


def _tc_first(x, Wl, Wr, b):
    def body(x_ref, wl_ref, wr_ref, b_ref, ua_ref, ub_ref, v_ref):
        xb = x_ref[...]
        u = jnp.dot(xb, wl_ref[...], preferred_element_type=jnp.float32)
        ua_ref[...] = u[:, :H]
        ub_ref[...] = u[:, H:]
        v_ref[...] = jnp.dot(xb, wr_ref[...],
                             preferred_element_type=jnp.float32) + b_ref[...]

    return pl.pallas_call(
        body,
        grid=(N // BN,),
        in_specs=[
            pl.BlockSpec((BN, D), lambda i: (i, 0)),
            pl.BlockSpec((D, D), lambda i: (0, 0)),
            pl.BlockSpec((D, D), lambda i: (0, 0)),
            pl.BlockSpec((1, D), lambda i: (0, 0)),
        ],
        out_specs=[
            pl.BlockSpec((BN, H), lambda i: (i, 0)),
            pl.BlockSpec((BN, H), lambda i: (i, 0)),
            pl.BlockSpec((BN, D), lambda i: (i, 0)),
        ],
        out_shape=[
            jax.ShapeDtypeStruct((N, H), jnp.float32),
            jax.ShapeDtypeStruct((N, H), jnp.float32),
            jax.ShapeDtypeStruct((N, D), jnp.float32),
        ],
    )(x, Wl, Wr, b.reshape(1, D))


def _tc_mid(aa, ab, deg, v, Wl, Wr, b):
    def body(aa_ref, ab_ref, deg_ref, v_ref, wl_ref, wr_ref, b_ref,
             ua_ref, ub_ref, vo_ref):
        d = jnp.maximum(deg_ref[...], 1.0)
        vb = v_ref[...]
        ha = jnp.maximum(aa_ref[...] / d + vb[:, :H], 0.0)
        hb = jnp.maximum(ab_ref[...] / d + vb[:, H:], 0.0)
        wl = wl_ref[...]
        wr = wr_ref[...]
        u = (jnp.dot(ha, wl[:H, :], preferred_element_type=jnp.float32)
             + jnp.dot(hb, wl[H:, :], preferred_element_type=jnp.float32))
        ua_ref[...] = u[:, :H]
        ub_ref[...] = u[:, H:]
        vo_ref[...] = (jnp.dot(ha, wr[:H, :], preferred_element_type=jnp.float32)
                       + jnp.dot(hb, wr[H:, :], preferred_element_type=jnp.float32)
                       + b_ref[...])

    return pl.pallas_call(
        body,
        grid=(N // BN,),
        in_specs=[
            pl.BlockSpec((BN, H), lambda i: (i, 0)),
            pl.BlockSpec((BN, H), lambda i: (i, 0)),
            pl.BlockSpec((BN, 1), lambda i: (i, 0)),
            pl.BlockSpec((BN, D), lambda i: (i, 0)),
            pl.BlockSpec((D, D), lambda i: (0, 0)),
            pl.BlockSpec((D, D), lambda i: (0, 0)),
            pl.BlockSpec((1, D), lambda i: (0, 0)),
        ],
        out_specs=[
            pl.BlockSpec((BN, H), lambda i: (i, 0)),
            pl.BlockSpec((BN, H), lambda i: (i, 0)),
            pl.BlockSpec((BN, D), lambda i: (i, 0)),
        ],
        out_shape=[
            jax.ShapeDtypeStruct((N, H), jnp.float32),
            jax.ShapeDtypeStruct((N, H), jnp.float32),
            jax.ShapeDtypeStruct((N, D), jnp.float32),
        ],
    )(aa, ab, deg, v, Wl, Wr, b.reshape(1, D))


def _tc_last(aa, ab, deg, v):
    def body(aa_ref, ab_ref, deg_ref, v_ref, oa_ref, ob_ref):
        d = jnp.maximum(deg_ref[...], 1.0)
        vb = v_ref[...]
        oa_ref[...] = aa_ref[...] / d + vb[:, :H]
        ob_ref[...] = ab_ref[...] / d + vb[:, H:]

    return pl.pallas_call(
        body,
        grid=(N // BN,),
        in_specs=[
            pl.BlockSpec((BN, H), lambda i: (i, 0)),
            pl.BlockSpec((BN, H), lambda i: (i, 0)),
            pl.BlockSpec((BN, 1), lambda i: (i, 0)),
            pl.BlockSpec((BN, D), lambda i: (i, 0)),
        ],
        out_specs=[
            pl.BlockSpec((BN, H), lambda i: (i, 0)),
            pl.BlockSpec((BN, H), lambda i: (i, 0)),
        ],
        out_shape=[
            jax.ShapeDtypeStruct((N, H), jnp.float32),
            jax.ShapeDtypeStruct((N, H), jnp.float32),
        ],
    )(aa, ab, deg, v)


def kernel(x, edge_index, Wl0, Wr0, b0, Wl1, Wr1, b1, Wl2, Wr2, b2):
    src = edge_index[0]
    dst = edge_index[1]
    pad = E_PAD - E
    src_p = jnp.concatenate([src, jnp.zeros((pad,), jnp.int32)])
    dst_p = jnp.concatenate([dst, jnp.full((pad,), N, jnp.int32)])
    src2 = src_p.reshape(E_PAD // CHUNK, CHUNK)
    dst2 = dst_p.reshape(E_PAD // CHUNK, CHUNK)
    zeros2 = jnp.zeros((ACC_ROWS, H), jnp.float32)
    zeros1 = jnp.zeros((ACC_ROWS,), jnp.float32)

    ua, ub, v = _tc_first(x, Wl0, Wr0, b0)
    aa, ab, deg_t = _sc_segsum(ua, ub, src2, dst2, zeros2, zeros1, with_deg=True)
    deg = deg_t.reshape(-1)[:N].reshape(N, 1)

    ua, ub, v = _tc_mid(aa[:N], ab[:N], deg, v, Wl1, Wr1, b1)
    aa, ab = _sc_segsum(ua, ub, src2, dst2, zeros2, zeros1, with_deg=False)

    ua, ub, v = _tc_mid(aa[:N], ab[:N], deg, v, Wl2, Wr2, b2)
    aa, ab = _sc_segsum(ua, ub, src2, dst2, zeros2, zeros1, with_deg=False)

    oa, ob = _tc_last(aa[:N], ab[:N], deg, v)
    return jnp.concatenate([oa, ob], axis=1)


# speedup vs baseline: 8.8844x; 1.4630x over previous
import functools

import jax
import jax.numpy as jnp
from jax import lax
from jax.experimental import pallas as pl
from jax.experimental.pallas import tpu as pltpu
from jax.experimental.pallas import tpu_sc as plsc

N = 100000
E = 1600000
D = 32
H = 16

NC = 2
NS = 16

CHUNK = 128
GROUP = 16
TILE_E = 102400
E_PAD = NS * TILE_E
GROUPS = TILE_E // (GROUP * CHUNK)
RING = 8
DEPTH = 6

ACC_ROWS = 100864
ZROWS = 788

def _sc_segsum(ua, ub, src2, dst2, zeros2, zeros1, with_deg):
    out_type = [
        jax.ShapeDtypeStruct((ACC_ROWS, H), jnp.float32),
        jax.ShapeDtypeStruct((ACC_ROWS, H), jnp.float32),
    ]
    scratch = [
        pltpu.VMEM_SHARED((ACC_ROWS, H), jnp.float32),
        pltpu.VMEM((GROUP, CHUNK), jnp.int32),
        pltpu.VMEM((GROUP, CHUNK), jnp.int32),
        pltpu.VMEM((RING, CHUNK, H), jnp.float32),
        pltpu.SemaphoreType.DMA,
    ]
    if with_deg:
        out_type.append(jax.ShapeDtypeStruct((NS, ACC_ROWS // NS), jnp.float32))
        scratch += [
            pltpu.VMEM_SHARED((ACC_ROWS,), jnp.float32),
            pltpu.VMEM((CHUNK,), jnp.float32),
        ]

    mesh = plsc.VectorSubcoreMesh(core_axis_name="c", subcore_axis_name="s")

    def body(ua_hbm, ub_hbm, src_hbm, dst_hbm, z2_hbm, z1_hbm,
             oa_hbm, ob_hbm, *rest):
        if with_deg:
            deg_hbm, acc, idx_s, idx_d, rows, gsem, deg_acc, ones = rest
        else:
            acc, idx_s, idx_d, rows, gsem = rest
        c = lax.axis_index("c")
        s = lax.axis_index("s")

        nrows = ACC_ROWS // NS
        zbase = s * nrows
        pltpu.sync_copy(z2_hbm.at[pl.ds(zbase, nrows)],
                        acc.at[pl.ds(zbase, nrows)])
        if with_deg:
            pltpu.sync_copy(z1_hbm.at[pl.ds(zbase, nrows)],
                            deg_acc.at[pl.ds(zbase, nrows)])
            def fo(i, _):
                ones[pl.ds(i * 16, 16)] = jnp.ones((16,), jnp.float32)
                return 0
            lax.fori_loop(0, CHUNK // 16, fo, 0)
        plsc.subcore_barrier()

        rows_per_tile = TILE_E // CHUNK
        def fire(j):
            slot = j % RING
            @pl.when(c == 0)
            def _():
                pltpu.async_copy(ua_hbm.at[idx_s.at[j]], rows.at[slot], gsem)
            @pl.when(c == 1)
            def _():
                pltpu.async_copy(ub_hbm.at[idx_s.at[j]], rows.at[slot], gsem)

        def group_body(g, _):
            r0 = s * rows_per_tile + g * GROUP
            pltpu.sync_copy(src_hbm.at[pl.ds(r0, GROUP)], idx_s)
            pltpu.sync_copy(dst_hbm.at[pl.ds(r0, GROUP)], idx_d)
            for j in range(DEPTH):
                fire(j)
            for j in range(GROUP):
                pltpu.make_async_copy(ua_hbm.at[idx_s.at[j]],
                                      rows.at[j % RING], gsem).wait()
                pltpu.sync_copy(rows.at[j % RING], acc.at[idx_d.at[j]], add=True)
                if with_deg:
                    @pl.when(c == 0)
                    def _():
                        pltpu.sync_copy(ones, deg_acc.at[idx_d.at[j]], add=True)
                if j + DEPTH < GROUP:
                    fire(j + DEPTH)
            return 0
        lax.fori_loop(0, GROUPS, group_body, 0)

        plsc.subcore_barrier()
        @pl.when(c == 0)
        def _():
            pltpu.sync_copy(acc.at[pl.ds(zbase, nrows)],
                            oa_hbm.at[pl.ds(zbase, nrows)])
            if with_deg:
                pltpu.sync_copy(deg_acc.at[pl.ds(zbase, nrows)],
                                deg_hbm.at[s])
        @pl.when(c == 1)
        def _():
            pltpu.sync_copy(acc.at[pl.ds(zbase, nrows)],
                            ob_hbm.at[pl.ds(zbase, nrows)])

    f = pl.kernel(body, out_type=tuple(out_type), mesh=mesh,
                  scratch_types=tuple(scratch),
                  compiler_params=pltpu.CompilerParams(use_tc_tiling_on_sc=False))
    return f(ua, ub, src2, dst2, zeros2, zeros1)


BN = 2000


def _tc_first(x, Wl, Wr, b):
    def body(x_ref, wl_ref, wr_ref, b_ref, ua_ref, ub_ref, v_ref):
        xb = x_ref[...]
        u = jnp.dot(xb, wl_ref[...], preferred_element_type=jnp.float32)
        ua_ref[...] = u[:, :H]
        ub_ref[...] = u[:, H:]
        v_ref[...] = jnp.dot(xb, wr_ref[...],
                             preferred_element_type=jnp.float32) + b_ref[...]

    return pl.pallas_call(
        body,
        grid=(N // BN,),
        in_specs=[
            pl.BlockSpec((BN, D), lambda i: (i, 0)),
            pl.BlockSpec((D, D), lambda i: (0, 0)),
            pl.BlockSpec((D, D), lambda i: (0, 0)),
            pl.BlockSpec((1, D), lambda i: (0, 0)),
        ],
        out_specs=[
            pl.BlockSpec((BN, H), lambda i: (i, 0)),
            pl.BlockSpec((BN, H), lambda i: (i, 0)),
            pl.BlockSpec((BN, D), lambda i: (i, 0)),
        ],
        out_shape=[
            jax.ShapeDtypeStruct((N, H), jnp.float32),
            jax.ShapeDtypeStruct((N, H), jnp.float32),
            jax.ShapeDtypeStruct((N, D), jnp.float32),
        ],
    )(x, Wl, Wr, b.reshape(1, D))


def _tc_mid(aa, ab, deg, v, Wl, Wr, b):
    def body(aa_ref, ab_ref, deg_ref, v_ref, wl_ref, wr_ref, b_ref,
             ua_ref, ub_ref, vo_ref):
        d = jnp.maximum(deg_ref[...], 1.0)
        vb = v_ref[...]
        ha = jnp.maximum(aa_ref[...] / d + vb[:, :H], 0.0)
        hb = jnp.maximum(ab_ref[...] / d + vb[:, H:], 0.0)
        wl = wl_ref[...]
        wr = wr_ref[...]
        u = (jnp.dot(ha, wl[:H, :], preferred_element_type=jnp.float32)
             + jnp.dot(hb, wl[H:, :], preferred_element_type=jnp.float32))
        ua_ref[...] = u[:, :H]
        ub_ref[...] = u[:, H:]
        vo_ref[...] = (jnp.dot(ha, wr[:H, :], preferred_element_type=jnp.float32)
                       + jnp.dot(hb, wr[H:, :], preferred_element_type=jnp.float32)
                       + b_ref[...])

    return pl.pallas_call(
        body,
        grid=(N // BN,),
        in_specs=[
            pl.BlockSpec((BN, H), lambda i: (i, 0)),
            pl.BlockSpec((BN, H), lambda i: (i, 0)),
            pl.BlockSpec((BN, 1), lambda i: (i, 0)),
            pl.BlockSpec((BN, D), lambda i: (i, 0)),
            pl.BlockSpec((D, D), lambda i: (0, 0)),
            pl.BlockSpec((D, D), lambda i: (0, 0)),
            pl.BlockSpec((1, D), lambda i: (0, 0)),
        ],
        out_specs=[
            pl.BlockSpec((BN, H), lambda i: (i, 0)),
            pl.BlockSpec((BN, H), lambda i: (i, 0)),
            pl.BlockSpec((BN, D), lambda i: (i, 0)),
        ],
        out_shape=[
            jax.ShapeDtypeStruct((N, H), jnp.float32),
            jax.ShapeDtypeStruct((N, H), jnp.float32),
            jax.ShapeDtypeStruct((N, D), jnp.float32),
        ],
    )(aa, ab, deg, v, Wl, Wr, b.reshape(1, D))


def _tc_last(aa, ab, deg, v):
    def body(aa_ref, ab_ref, deg_ref, v_ref, oa_ref, ob_ref):
        d = jnp.maximum(deg_ref[...], 1.0)
        vb = v_ref[...]
        oa_ref[...] = aa_ref[...] / d + vb[:, :H]
        ob_ref[...] = ab_ref[...] / d + vb[:, H:]

    return pl.pallas_call(
        body,
        grid=(N // BN,),
        in_specs=[
            pl.BlockSpec((BN, H), lambda i: (i, 0)),
            pl.BlockSpec((BN, H), lambda i: (i, 0)),
            pl.BlockSpec((BN, 1), lambda i: (i, 0)),
            pl.BlockSpec((BN, D), lambda i: (i, 0)),
        ],
        out_specs=[
            pl.BlockSpec((BN, H), lambda i: (i, 0)),
            pl.BlockSpec((BN, H), lambda i: (i, 0)),
        ],
        out_shape=[
            jax.ShapeDtypeStruct((N, H), jnp.float32),
            jax.ShapeDtypeStruct((N, H), jnp.float32),
        ],
    )(aa, ab, deg, v)


def kernel(x, edge_index, Wl0, Wr0, b0, Wl1, Wr1, b1, Wl2, Wr2, b2):
    src = edge_index[0]
    dst = edge_index[1]
    pad = E_PAD - E
    src_p = jnp.concatenate([src, jnp.zeros((pad,), jnp.int32)])
    dst_p = jnp.concatenate([dst, jnp.full((pad,), N, jnp.int32)])
    src2 = src_p.reshape(E_PAD // CHUNK, CHUNK)
    dst2 = dst_p.reshape(E_PAD // CHUNK, CHUNK)
    zeros2 = jnp.zeros((ACC_ROWS, H), jnp.float32)
    zeros1 = jnp.zeros((ACC_ROWS,), jnp.float32)

    ua, ub, v = _tc_first(x, Wl0, Wr0, b0)
    aa, ab, deg_t = _sc_segsum(ua, ub, src2, dst2, zeros2, zeros1, with_deg=True)
    deg = deg_t.reshape(-1)[:N].reshape(N, 1)

    ua, ub, v = _tc_mid(aa[:N], ab[:N], deg, v, Wl1, Wr1, b1)
    aa, ab = _sc_segsum(ua, ub, src2, dst2, zeros2, zeros1, with_deg=False)

    ua, ub, v = _tc_mid(aa[:N], ab[:N], deg, v, Wl2, Wr2, b2)
    aa, ab = _sc_segsum(ua, ub, src2, dst2, zeros2, zeros1, with_deg=False)

    oa, ob = _tc_last(aa[:N], ab[:N], deg, v)
    return jnp.concatenate([oa, ob], axis=1)
